# Initial kernel scaffold; baseline (speedup 1.0000x reference)
import jax
import jax.numpy as jnp
from jax.experimental import pallas as pl


def kernel(feat, W_proj, b_proj, emb_table, node_ids, ntype_ids):
    raise NotImplementedError("write your pallas kernel here")


# speedup vs baseline: 3.1052x; 3.1052x over previous
import functools

import jax
import jax.numpy as jnp
from jax import lax
from jax.experimental import pallas as pl
from jax.experimental.pallas import tpu as pltpu
from jax.experimental.pallas import tpu_sc as plsc

B = 16384
D_FEAT = 512
EMBED = 512

NC = 2
NS = 16
NW = NC * NS
B_PER_W = B // NW
CHUNK = 64
N_CHUNKS = B_PER_W // CHUNK

@functools.cache
def _make_sc_gather():
    mesh = plsc.VectorSubcoreMesh(core_axis_name="c", subcore_axis_name="s")

    @functools.partial(
        pl.kernel,
        out_type=(
            jax.ShapeDtypeStruct((B, D_FEAT), jnp.float32),
            jax.ShapeDtypeStruct((B, EMBED), jnp.float32),
        ),
        mesh=mesh,
        scratch_types=[
            pltpu.VMEM((N_CHUNKS, CHUNK), jnp.int32),
            pltpu.VMEM((CHUNK, D_FEAT), jnp.float32),
            pltpu.VMEM((CHUNK, EMBED), jnp.float32),
            pltpu.SemaphoreType.DMA,
            pltpu.SemaphoreType.DMA,
        ],
    )
    def _sc_gather(feat_hbm, emb_hbm, ids_hbm, g_out, l_out,
                   idx_v, rows_f, rows_e, sem_f, sem_e):
        wid = lax.axis_index("s") * NC + lax.axis_index("c")
        base = wid * B_PER_W
        pltpu.sync_copy(ids_hbm.at[wid], idx_v)
        for c in range(N_CHUNKS):
            cf = pltpu.async_copy(feat_hbm.at[idx_v.at[c]], rows_f, sem_f)
            ce = pltpu.async_copy(emb_hbm.at[idx_v.at[c]], rows_e, sem_e)
            cf.wait()
            pltpu.sync_copy(rows_f, g_out.at[pl.ds(base + c * CHUNK, CHUNK)])
            ce.wait()
            pltpu.sync_copy(rows_e, l_out.at[pl.ds(base + c * CHUNK, CHUNK)])

    return _sc_gather


BM = 512


def _tc_body(ntype_ref, g_ref, l_ref, w_ref, b_ref, o_ref):
    proj = lax.dot_general(
        g_ref[...], w_ref[...],
        dimension_numbers=(((1,), (1,)), ((), ())),
        preferred_element_type=jnp.float32,
    ) + b_ref[...]
    o_ref[...] = jnp.where(ntype_ref[...] == 0, proj, l_ref[...])


def _tc_combine(ntype2d, g, l, w, b2d):
    return pl.pallas_call(
        _tc_body,
        grid=(B // BM,),
        in_specs=[
            pl.BlockSpec((BM, 1), lambda i: (i, 0)),
            pl.BlockSpec((BM, D_FEAT), lambda i: (i, 0)),
            pl.BlockSpec((BM, EMBED), lambda i: (i, 0)),
            pl.BlockSpec((EMBED, D_FEAT), lambda i: (0, 0)),
            pl.BlockSpec((1, EMBED), lambda i: (0, 0)),
        ],
        out_specs=pl.BlockSpec((BM, EMBED), lambda i: (i, 0)),
        out_shape=jax.ShapeDtypeStruct((B, EMBED), jnp.float32),
    )(ntype2d, g, l, w, b2d)


def kernel(feat, W_proj, b_proj, emb_table, node_ids, ntype_ids):
    ids = node_ids.astype(jnp.int32).reshape(NW, N_CHUNKS, CHUNK)
    g, l = _make_sc_gather()(feat, emb_table, ids)
    ntype2d = ntype_ids.astype(jnp.int32).reshape(B, 1)
    return _tc_combine(ntype2d, g, l, W_proj, b_proj.reshape(1, EMBED))
